# baseline (speedup 1.0000x reference)
import functools

import jax
import jax.numpy as jnp
from jax import lax
from jax.experimental import pallas as pl
from jax.experimental.pallas import tpu as pltpu

E = 16
TOPK = 2
G = 8
KG = 4
ROUTE_SCALE = 2.5
D = 1024
F = 512
T = 1024

_NEG = -1e30


def _gate_kernel(x_ref, gw_ref, comb_ref):
    scores = jax.nn.sigmoid(
        lax.dot_general(x_ref[...], gw_ref[...], (((1,), (1,)), ((), ())),
                        preferred_element_type=jnp.float32))
    gcols = [jnp.maximum(scores[:, 2 * g:2 * g + 1], scores[:, 2 * g + 1:2 * g + 2])
             for g in range(G)]
    gs = jnp.concatenate(gcols, axis=1)
    iota_g = lax.broadcasted_iota(jnp.int32, (T, G), 1)
    keep = jnp.zeros((T, G), jnp.float32)
    work = gs
    for _ in range(KG):
        m = jnp.max(work, axis=1, keepdims=True)
        first = jnp.min(jnp.where(work == m, iota_g, G), axis=1, keepdims=True)
        sel = iota_g == first
        keep = keep + jnp.where(sel, 1.0, 0.0)
        work = jnp.where(sel, _NEG, work)
    keep_e = jnp.concatenate([keep[:, g:g + 1] for g in range(G) for _ in (0, 1)],
                             axis=1)
    masked = jnp.where(keep_e > 0.5, scores, _NEG)
    iota_e = lax.broadcasted_iota(jnp.int32, (T, E), 1)
    comb = jnp.zeros((T, E), jnp.float32)
    wsum = jnp.zeros((T, 1), jnp.float32)
    work = masked
    for _ in range(TOPK):
        m = jnp.max(work, axis=1, keepdims=True)
        first = jnp.min(jnp.where(work == m, iota_e, E), axis=1, keepdims=True)
        sel = iota_e == first
        comb = comb + jnp.where(sel, scores, 0.0)
        wsum = wsum + m
        work = jnp.where(sel, _NEG, work)
    comb_ref[...] = comb * (ROUTE_SCALE / (wsum + 1e-20))


def _moe_kernel(x_ref, w1_ref, w3_ref, w2_ref, comb_ref, y_ref):
    e = pl.program_id(0)
    x = x_ref[...].astype(jnp.bfloat16)
    w1 = w1_ref[0].astype(jnp.bfloat16)
    w3 = w3_ref[0].astype(jnp.bfloat16)
    w2 = w2_ref[0].astype(jnp.bfloat16)
    h1 = lax.dot_general(x, w1, (((1,), (1,)), ((), ())),
                         preferred_element_type=jnp.float32)
    h3 = lax.dot_general(x, w3, (((1,), (1,)), ((), ())),
                         preferred_element_type=jnp.float32)
    act = (h1 * jax.nn.sigmoid(h1) * h3).astype(jnp.bfloat16)
    out = lax.dot_general(act, w2, (((1,), (1,)), ((), ())),
                          preferred_element_type=jnp.float32)
    c = jnp.zeros((T, 1), jnp.float32)
    for j in range(E + 1):
        c = c + jnp.where(e == j, comb_ref[:, j:j + 1], 0.0)

    @pl.when(e == 0)
    def _():
        y_ref[...] = c * out

    @pl.when(e > 0)
    def _():
        y_ref[...] = y_ref[...] + c * out


@jax.jit
def kernel(x, gate_w, W1, W2, W3, Ws1, Ws2, Ws3):
    comb = pl.pallas_call(
        _gate_kernel,
        out_shape=jax.ShapeDtypeStruct((T, E), jnp.float32),
    )(x, gate_w)
    comb17 = jnp.concatenate([comb, jnp.ones((T, 1), jnp.float32)], axis=1)

    W1s = jnp.concatenate([W1, Ws1[None]], axis=0)
    W3s = jnp.concatenate([W3, Ws3[None]], axis=0)
    W2s = jnp.concatenate([W2, Ws2[None]], axis=0)

    y = pl.pallas_call(
        _moe_kernel,
        grid=(E + 1,),
        in_specs=[
            pl.BlockSpec((T, D), lambda e: (0, 0)),
            pl.BlockSpec((1, F, D), lambda e: (e, 0, 0)),
            pl.BlockSpec((1, F, D), lambda e: (e, 0, 0)),
            pl.BlockSpec((1, D, F), lambda e: (e, 0, 0)),
            pl.BlockSpec((T, E + 1), lambda e: (0, 0)),
        ],
        out_specs=pl.BlockSpec((T, D), lambda e: (0, 0)),
        out_shape=jax.ShapeDtypeStruct((T, D), jnp.float32),
        compiler_params=pltpu.CompilerParams(
            dimension_semantics=("arbitrary",)),
    )(x, W1s, W3s, W2s, comb17)
    return y


# speedup vs baseline: 1.8864x; 1.8864x over previous
import jax
import jax.numpy as jnp
from jax import lax
from jax.experimental import pallas as pl
from jax.experimental.pallas import tpu as pltpu

E = 16
TOPK = 2
G = 8
KG = 4
ROUTE_SCALE = 2.5
D = 1024
F = 512
T = 1024

_NEG = -1e30


def _gate_kernel(x_ref, gw_ref, comb_ref, xbf_ref):
    x = x_ref[...]
    xbf_ref[...] = x.astype(jnp.bfloat16)
    scores = jax.nn.sigmoid(
        lax.dot_general(x, gw_ref[...], (((1,), (1,)), ((), ())),
                        preferred_element_type=jnp.float32))
    gcols = [jnp.maximum(scores[:, 2 * g:2 * g + 1], scores[:, 2 * g + 1:2 * g + 2])
             for g in range(G)]
    gs = jnp.concatenate(gcols, axis=1)
    iota_g = lax.broadcasted_iota(jnp.int32, (T, G), 1)
    keep = jnp.zeros((T, G), jnp.float32)
    work = gs
    for _ in range(KG):
        m = jnp.max(work, axis=1, keepdims=True)
        first = jnp.min(jnp.where(work == m, iota_g, G), axis=1, keepdims=True)
        sel = iota_g == first
        keep = keep + jnp.where(sel, 1.0, 0.0)
        work = jnp.where(sel, _NEG, work)
    keep_e = jnp.concatenate([keep[:, g:g + 1] for g in range(G) for _ in (0, 1)],
                             axis=1)
    masked = jnp.where(keep_e > 0.5, scores, _NEG)
    iota_e = lax.broadcasted_iota(jnp.int32, (T, E), 1)
    comb = jnp.zeros((T, E), jnp.float32)
    wsum = jnp.zeros((T, 1), jnp.float32)
    work = masked
    for _ in range(TOPK):
        m = jnp.max(work, axis=1, keepdims=True)
        first = jnp.min(jnp.where(work == m, iota_e, E), axis=1, keepdims=True)
        sel = iota_e == first
        comb = comb + jnp.where(sel, scores, 0.0)
        wsum = wsum + m
        work = jnp.where(sel, _NEG, work)
    comb_ref[...] = comb * (ROUTE_SCALE / (wsum + 1e-20))


def _mlp(xbf, w1, w3, w2):
    h1 = lax.dot_general(xbf, w1.astype(jnp.bfloat16), (((1,), (1,)), ((), ())),
                         preferred_element_type=jnp.float32)
    h3 = lax.dot_general(xbf, w3.astype(jnp.bfloat16), (((1,), (1,)), ((), ())),
                         preferred_element_type=jnp.float32)
    act = (h1 * jax.nn.sigmoid(h1) * h3).astype(jnp.bfloat16)
    return lax.dot_general(act, w2.astype(jnp.bfloat16), (((1,), (1,)), ((), ())),
                           preferred_element_type=jnp.float32)


def _moe_kernel(xbf_ref, w1_ref, w3_ref, w2_ref, ws1_ref, ws3_ref, ws2_ref,
                comb_ref, y_ref):
    e = pl.program_id(0)
    xbf = xbf_ref[...]
    out = _mlp(xbf, w1_ref[0], w3_ref[0], w2_ref[0])
    onehot = (lax.broadcasted_iota(jnp.int32, (E, 1), 0) == e).astype(jnp.float32)
    c = lax.dot_general(comb_ref[...], onehot, (((1,), (0,)), ((), ())),
                        preferred_element_type=jnp.float32)

    @pl.when(e == 0)
    def _():
        y_ref[...] = c * out + _mlp(xbf, ws1_ref[...], ws3_ref[...], ws2_ref[...])

    @pl.when(e > 0)
    def _():
        y_ref[...] = y_ref[...] + c * out


@jax.jit
def kernel(x, gate_w, W1, W2, W3, Ws1, Ws2, Ws3):
    comb, xbf = pl.pallas_call(
        _gate_kernel,
        out_shape=(jax.ShapeDtypeStruct((T, E), jnp.float32),
                   jax.ShapeDtypeStruct((T, D), jnp.bfloat16)),
    )(x, gate_w)

    y = pl.pallas_call(
        _moe_kernel,
        grid=(E,),
        in_specs=[
            pl.BlockSpec((T, D), lambda e: (0, 0)),
            pl.BlockSpec((1, F, D), lambda e: (e, 0, 0)),
            pl.BlockSpec((1, F, D), lambda e: (e, 0, 0)),
            pl.BlockSpec((1, D, F), lambda e: (e, 0, 0)),
            pl.BlockSpec((F, D), lambda e: (0, 0)),
            pl.BlockSpec((F, D), lambda e: (0, 0)),
            pl.BlockSpec((D, F), lambda e: (0, 0)),
            pl.BlockSpec((T, E), lambda e: (0, 0)),
        ],
        out_specs=pl.BlockSpec((T, D), lambda e: (0, 0)),
        out_shape=jax.ShapeDtypeStruct((T, D), jnp.float32),
        compiler_params=pltpu.CompilerParams(
            dimension_semantics=("arbitrary",)),
    )(xbf, W1, W3, W2, Ws1, Ws3, Ws2, comb)
    return y


# speedup vs baseline: 1.9904x; 1.0551x over previous
import jax
import jax.numpy as jnp
from jax import lax
from jax.experimental import pallas as pl
from jax.experimental.pallas import tpu as pltpu

E = 16
TOPK = 2
G = 8
KG = 4
ROUTE_SCALE = 2.5
D = 1024
F = 512
T = 1024

_NEG = -1e30


def _gate_kernel(x_ref, gw_ref, comb_ref, xbf_ref):
    x = x_ref[...]
    xbf_ref[...] = x.astype(jnp.bfloat16)
    scores = jax.nn.sigmoid(
        lax.dot_general(gw_ref[...], x, (((1,), (1,)), ((), ())),
                        preferred_element_type=jnp.float32))
    grows = [jnp.maximum(scores[2 * g:2 * g + 1, :], scores[2 * g + 1:2 * g + 2, :])
             for g in range(G)]
    gs = jnp.concatenate(grows, axis=0)
    iota_g = lax.broadcasted_iota(jnp.int32, (G, T), 0)
    keep = jnp.zeros((G, T), jnp.float32)
    work = gs
    for _ in range(KG):
        m = jnp.max(work, axis=0, keepdims=True)
        first = jnp.min(jnp.where(work == m, iota_g, G), axis=0, keepdims=True)
        sel = iota_g == first
        keep = keep + jnp.where(sel, 1.0, 0.0)
        work = jnp.where(sel, _NEG, work)
    keep_e = jnp.concatenate([keep[g:g + 1, :] for g in range(G) for _ in (0, 1)],
                             axis=0)
    masked = jnp.where(keep_e > 0.5, scores, _NEG)
    iota_e = lax.broadcasted_iota(jnp.int32, (E, T), 0)
    comb = jnp.zeros((E, T), jnp.float32)
    wsum = jnp.zeros((1, T), jnp.float32)
    work = masked
    for _ in range(TOPK):
        m = jnp.max(work, axis=0, keepdims=True)
        first = jnp.min(jnp.where(work == m, iota_e, E), axis=0, keepdims=True)
        sel = iota_e == first
        comb = comb + jnp.where(sel, scores, 0.0)
        wsum = wsum + m
        work = jnp.where(sel, _NEG, work)
    comb_ref[...] = jnp.transpose(comb * (ROUTE_SCALE / (wsum + 1e-20)))


def _mlp(xbf, w1, w3, w2):
    h1 = lax.dot_general(xbf, w1.astype(jnp.bfloat16), (((1,), (1,)), ((), ())),
                         preferred_element_type=jnp.float32)
    h3 = lax.dot_general(xbf, w3.astype(jnp.bfloat16), (((1,), (1,)), ((), ())),
                         preferred_element_type=jnp.float32)
    act = (h1 * jax.nn.sigmoid(h1) * h3).astype(jnp.bfloat16)
    return lax.dot_general(act, w2.astype(jnp.bfloat16), (((1,), (1,)), ((), ())),
                           preferred_element_type=jnp.float32)


def _moe_kernel(xbf_ref, w1_ref, w3_ref, w2_ref, ws1_ref, ws3_ref, ws2_ref,
                comb_ref, y_ref):
    e = pl.program_id(0)
    xbf = xbf_ref[...]
    out = _mlp(xbf, w1_ref[0], w3_ref[0], w2_ref[0])
    onehot = (lax.broadcasted_iota(jnp.int32, (E, 1), 0) == e).astype(jnp.float32)
    c = lax.dot_general(comb_ref[...], onehot, (((1,), (0,)), ((), ())),
                        preferred_element_type=jnp.float32)

    @pl.when(e == 0)
    def _():
        y_ref[...] = c * out + _mlp(xbf, ws1_ref[...], ws3_ref[...], ws2_ref[...])

    @pl.when(e > 0)
    def _():
        y_ref[...] = y_ref[...] + c * out


@jax.jit
def kernel(x, gate_w, W1, W2, W3, Ws1, Ws2, Ws3):
    comb, xbf = pl.pallas_call(
        _gate_kernel,
        out_shape=(jax.ShapeDtypeStruct((T, E), jnp.float32),
                   jax.ShapeDtypeStruct((T, D), jnp.bfloat16)),
    )(x, gate_w)

    y = pl.pallas_call(
        _moe_kernel,
        grid=(E,),
        in_specs=[
            pl.BlockSpec((T, D), lambda e: (0, 0)),
            pl.BlockSpec((1, F, D), lambda e: (e, 0, 0)),
            pl.BlockSpec((1, F, D), lambda e: (e, 0, 0)),
            pl.BlockSpec((1, D, F), lambda e: (e, 0, 0)),
            pl.BlockSpec((F, D), lambda e: (0, 0)),
            pl.BlockSpec((F, D), lambda e: (0, 0)),
            pl.BlockSpec((D, F), lambda e: (0, 0)),
            pl.BlockSpec((T, E), lambda e: (0, 0)),
        ],
        out_specs=pl.BlockSpec((T, D), lambda e: (0, 0)),
        out_shape=jax.ShapeDtypeStruct((T, D), jnp.float32),
        compiler_params=pltpu.CompilerParams(
            dimension_semantics=("arbitrary",)),
    )(xbf, W1, W3, W2, Ws1, Ws3, Ws2, comb)
    return y


# speedup vs baseline: 2.0048x; 1.0072x over previous
import jax
import jax.numpy as jnp
from jax import lax
from jax.experimental import pallas as pl
from jax.experimental.pallas import tpu as pltpu

E = 16
TOPK = 2
G = 8
KG = 4
ROUTE_SCALE = 2.5
D = 1024
F = 512
T = 1024

_NEG = -1e30


def _gate_kernel(x_ref, gw_ref, comb_ref, xbf_ref):
    x = x_ref[...]
    xbf_ref[...] = x.astype(jnp.bfloat16)
    scores = jax.nn.sigmoid(
        lax.dot_general(gw_ref[...], x, (((1,), (1,)), ((), ())),
                        preferred_element_type=jnp.float32))
    grows = [jnp.maximum(scores[2 * g:2 * g + 1, :], scores[2 * g + 1:2 * g + 2, :])
             for g in range(G)]
    gs = jnp.concatenate(grows, axis=0)
    iota_g = lax.broadcasted_iota(jnp.int32, (G, T), 0)
    keep = jnp.zeros((G, T), jnp.float32)
    work = gs
    for _ in range(KG):
        m = jnp.max(work, axis=0, keepdims=True)
        first = jnp.min(jnp.where(work == m, iota_g, G), axis=0, keepdims=True)
        sel = iota_g == first
        keep = keep + jnp.where(sel, 1.0, 0.0)
        work = jnp.where(sel, _NEG, work)
    keep_e = jnp.concatenate([keep[g:g + 1, :] for g in range(G) for _ in (0, 1)],
                             axis=0)
    masked = jnp.where(keep_e > 0.5, scores, _NEG)
    iota_e = lax.broadcasted_iota(jnp.int32, (E, T), 0)
    comb = jnp.zeros((E, T), jnp.float32)
    wsum = jnp.zeros((1, T), jnp.float32)
    work = masked
    for _ in range(TOPK):
        m = jnp.max(work, axis=0, keepdims=True)
        first = jnp.min(jnp.where(work == m, iota_e, E), axis=0, keepdims=True)
        sel = iota_e == first
        comb = comb + jnp.where(sel, scores, 0.0)
        wsum = wsum + m
        work = jnp.where(sel, _NEG, work)
    comb_ref[...] = jnp.transpose(comb * (ROUTE_SCALE / (wsum + 1e-20)))


def _mlp(xbf, w1, w3, w2):
    xf = xbf.astype(jnp.float32)
    h1 = lax.dot_general(xf, w1, (((1,), (1,)), ((), ())),
                         preferred_element_type=jnp.float32)
    h3 = lax.dot_general(xf, w3, (((1,), (1,)), ((), ())),
                         preferred_element_type=jnp.float32)
    act = h1 * jax.nn.sigmoid(h1) * h3
    return lax.dot_general(act, w2, (((1,), (1,)), ((), ())),
                           preferred_element_type=jnp.float32)


def _moe_kernel(xbf_ref, w1_ref, w3_ref, w2_ref, ws1_ref, ws3_ref, ws2_ref,
                comb_ref, y_ref):
    e = pl.program_id(0)
    xbf = xbf_ref[...]
    out = _mlp(xbf, w1_ref[0], w3_ref[0], w2_ref[0])
    onehot = (lax.broadcasted_iota(jnp.int32, (E, 1), 0) == e).astype(jnp.float32)
    c = lax.dot_general(comb_ref[...], onehot, (((1,), (0,)), ((), ())),
                        preferred_element_type=jnp.float32)

    @pl.when(e == 0)
    def _():
        y_ref[...] = c * out + _mlp(xbf, ws1_ref[...], ws3_ref[...], ws2_ref[...])

    @pl.when(e > 0)
    def _():
        y_ref[...] = y_ref[...] + c * out


@jax.jit
def kernel(x, gate_w, W1, W2, W3, Ws1, Ws2, Ws3):
    comb, xbf = pl.pallas_call(
        _gate_kernel,
        out_shape=(jax.ShapeDtypeStruct((T, E), jnp.float32),
                   jax.ShapeDtypeStruct((T, D), jnp.bfloat16)),
    )(x, gate_w)

    y = pl.pallas_call(
        _moe_kernel,
        grid=(E,),
        in_specs=[
            pl.BlockSpec((T, D), lambda e: (0, 0)),
            pl.BlockSpec((1, F, D), lambda e: (e, 0, 0)),
            pl.BlockSpec((1, F, D), lambda e: (e, 0, 0)),
            pl.BlockSpec((1, D, F), lambda e: (e, 0, 0)),
            pl.BlockSpec((F, D), lambda e: (0, 0)),
            pl.BlockSpec((F, D), lambda e: (0, 0)),
            pl.BlockSpec((D, F), lambda e: (0, 0)),
            pl.BlockSpec((T, E), lambda e: (0, 0)),
        ],
        out_specs=pl.BlockSpec((T, D), lambda e: (0, 0)),
        out_shape=jax.ShapeDtypeStruct((T, D), jnp.float32),
        compiler_params=pltpu.CompilerParams(
            dimension_semantics=("arbitrary",)),
    )(xbf, W1, W3, W2, Ws1, Ws3, Ws2, comb)
    return y
